# baseline (speedup 1.0000x reference)
import jax
import jax.numpy as jnp
from jax import lax
from jax.experimental import pallas as pl
from jax.experimental.pallas import tpu as pltpu
from jax.experimental.pallas import tpu_sc as plsc

BATCH = 4096
SEQ_LEN = 2048
ALPHABET = 5
LANES = 16

NUM_CORES = 2
NUM_SUBCORES = 16
NUM_WORKERS = NUM_CORES * NUM_SUBCORES

TILE_R = 8
TILE_C = 1024
COLS_TILES = SEQ_LEN // TILE_C
NG = (BATCH // TILE_R) * COLS_TILES // NUM_WORKERS
BLOCKS_G = TILE_R * TILE_C // LANES
CBLK = TILE_C // LANES


def _sc_body(seq_hbm, tbl_hbm, out_hbm,
             seq0, seq1, out0, out1, tbl_v, si0, si1, so0, so1):
    wid = lax.axis_index("s") * NUM_CORES + lax.axis_index("c")
    g0 = wid * NG
    seqb, outb = (seq0, seq1), (out0, out1)
    sis, sos = (si0, si1), (so0, so1)

    pltpu.sync_copy(tbl_hbm, tbl_v)

    def tile_origin(g):
        gg = g0 + g
        r0 = (gg // COLS_TILES) * TILE_R
        c0 = (gg % COLS_TILES) * TILE_C
        return r0, c0

    def load(g, s):
        r0, c0 = tile_origin(g)
        pltpu.async_copy(
            seq_hbm.at[pl.ds(r0, TILE_R), pl.ds(c0, TILE_C)], seqb[s], sis[s])

    def wait_load(g, s):
        r0, c0 = tile_origin(g)
        pltpu.make_async_copy(
            seq_hbm.at[pl.ds(r0, TILE_R), pl.ds(c0, TILE_C)], seqb[s],
            sis[s]).wait()

    def store(g, s):
        r0, c0 = tile_origin(g)
        pltpu.async_copy(
            outb[s], out_hbm.at[:, pl.ds(r0, TILE_R), pl.ds(c0, TILE_C)],
            sos[s])

    def wait_store(g, s):
        r0, c0 = tile_origin(g)
        pltpu.make_async_copy(
            outb[s], out_hbm.at[:, pl.ds(r0, TILE_R), pl.ds(c0, TILE_C)],
            sos[s]).wait()

    def compute(sv, ov):
        @plsc.parallel_loop(0, TILE_R * TILE_C, step=LANES, unroll=8)
        def _(i):
            r = i >> 10
            c = i & (TILE_C - 1)
            sg = sv[r, pl.ds(c, LANES)]
            for k in range(ALPHABET):
                val = plsc.load_gather(tbl_v, [sg + (8 * k)])
                ov[k, r, pl.ds(c, LANES)] = val

    load(0, 0)
    load(1, 1)

    def pair_body(p, _):
        for s in range(2):
            g = 2 * p + s
            wait_load(g, s)

            @pl.when(g >= 2)
            def _():
                wait_store(g - 2, s)

            compute(seqb[s], outb[s])
            store(g, s)

            @pl.when(g + 2 < NG)
            def _():
                load(g + 2, s)
        return ()

    lax.fori_loop(0, NG // 2, pair_body, ())
    wait_store(NG - 2, 0)
    wait_store(NG - 1, 1)


@jax.jit
def _encode(seq, tblT_pad):
    mesh = plsc.VectorSubcoreMesh(core_axis_name="c", subcore_axis_name="s")
    run = pl.kernel(
        _sc_body,
        out_type=jax.ShapeDtypeStruct((ALPHABET, BATCH, SEQ_LEN), jnp.float32),
        mesh=mesh,
        compiler_params=pltpu.CompilerParams(needs_layout_passes=False, use_tc_tiling_on_sc=True),
        scratch_types=[
            pltpu.VMEM((TILE_R, TILE_C), jnp.int32),
            pltpu.VMEM((TILE_R, TILE_C), jnp.int32),
            pltpu.VMEM((ALPHABET, TILE_R, TILE_C), jnp.float32),
            pltpu.VMEM((ALPHABET, TILE_R, TILE_C), jnp.float32),
            pltpu.VMEM((64,), jnp.float32),
            pltpu.SemaphoreType.DMA,
            pltpu.SemaphoreType.DMA,
            pltpu.SemaphoreType.DMA,
            pltpu.SemaphoreType.DMA,
        ],
    )
    return run(seq, tblT_pad)


def kernel(sequences, onehot_matrix):
    seq = sequences.astype(jnp.int32)
    tblT = jnp.pad(onehot_matrix.T.astype(jnp.float32), ((0, 0), (0, 3)))
    tblT_pad = jnp.pad(tblT.reshape(-1), (0, 24))
    out5 = _encode(seq, tblT_pad)
    return jnp.transpose(out5, (1, 2, 0))


# speedup vs baseline: 1.0038x; 1.0038x over previous
import jax
import jax.numpy as jnp
from jax import lax
from jax.experimental import pallas as pl
from jax.experimental.pallas import tpu as pltpu
from jax.experimental.pallas import tpu_sc as plsc

BATCH = 4096
SEQ_LEN = 2048
ALPHABET = 5
LANES = 16

NUM_CORES = 2
NUM_SUBCORES = 16
NUM_WORKERS = NUM_CORES * NUM_SUBCORES

TILE_R = 8
TILE_C = 1024
COLS_TILES = SEQ_LEN // TILE_C
NG = (BATCH // TILE_R) * COLS_TILES // NUM_WORKERS
BLOCKS_G = TILE_R * TILE_C // LANES
CBLK = TILE_C // LANES


def _sc_body(seq_hbm, tbl_hbm, out_hbm,
             seq0, seq1, out0, out1, tbl_v, si0, si1, so0, so1):
    wid = lax.axis_index("s") * NUM_CORES + lax.axis_index("c")
    g0 = wid * NG
    seqb, outb = (seq0, seq1), (out0, out1)
    sis, sos = (si0, si1), (so0, so1)

    pltpu.sync_copy(tbl_hbm, tbl_v)

    def tile_origin(g):
        gg = g0 + g
        r0 = (gg // COLS_TILES) * TILE_R
        c0 = (gg % COLS_TILES) * TILE_C
        return r0, c0

    def load(g, s):
        r0, c0 = tile_origin(g)
        pltpu.async_copy(
            seq_hbm.at[pl.ds(r0, TILE_R), pl.ds(c0, TILE_C)], seqb[s], sis[s])

    def wait_load(g, s):
        r0, c0 = tile_origin(g)
        pltpu.make_async_copy(
            seq_hbm.at[pl.ds(r0, TILE_R), pl.ds(c0, TILE_C)], seqb[s],
            sis[s]).wait()

    def store(g, s):
        r0, c0 = tile_origin(g)
        pltpu.async_copy(
            outb[s], out_hbm.at[:, pl.ds(r0, TILE_R), pl.ds(c0, TILE_C)],
            sos[s])

    def wait_store(g, s):
        r0, c0 = tile_origin(g)
        pltpu.make_async_copy(
            outb[s], out_hbm.at[:, pl.ds(r0, TILE_R), pl.ds(c0, TILE_C)],
            sos[s]).wait()

    def compute(sv, ov):
        @plsc.parallel_loop(0, TILE_R * TILE_C, step=LANES, unroll=4)
        def _(i):
            r = i >> 10
            c = i & (TILE_C - 1)
            sg = sv[r, pl.ds(c, LANES)]
            for k in range(ALPHABET):
                val = plsc.load_gather(tbl_v, [sg + (8 * k)])
                ov[k, r, pl.ds(c, LANES)] = val

    load(0, 0)
    load(1, 1)

    def pair_body(p, _):
        for s in range(2):
            g = 2 * p + s
            wait_load(g, s)

            @pl.when(g >= 2)
            def _():
                wait_store(g - 2, s)

            compute(seqb[s], outb[s])
            store(g, s)

            @pl.when(g + 2 < NG)
            def _():
                load(g + 2, s)
        return ()

    lax.fori_loop(0, NG // 2, pair_body, ())
    wait_store(NG - 2, 0)
    wait_store(NG - 1, 1)


@jax.jit
def _encode(seq, tblT_pad):
    mesh = plsc.VectorSubcoreMesh(core_axis_name="c", subcore_axis_name="s")
    run = pl.kernel(
        _sc_body,
        out_type=jax.ShapeDtypeStruct((ALPHABET, BATCH, SEQ_LEN), jnp.float32),
        mesh=mesh,
        compiler_params=pltpu.CompilerParams(needs_layout_passes=False, use_tc_tiling_on_sc=True),
        scratch_types=[
            pltpu.VMEM((TILE_R, TILE_C), jnp.int32),
            pltpu.VMEM((TILE_R, TILE_C), jnp.int32),
            pltpu.VMEM((ALPHABET, TILE_R, TILE_C), jnp.float32),
            pltpu.VMEM((ALPHABET, TILE_R, TILE_C), jnp.float32),
            pltpu.VMEM((64,), jnp.float32),
            pltpu.SemaphoreType.DMA,
            pltpu.SemaphoreType.DMA,
            pltpu.SemaphoreType.DMA,
            pltpu.SemaphoreType.DMA,
        ],
    )
    return run(seq, tblT_pad)


def kernel(sequences, onehot_matrix):
    seq = sequences.astype(jnp.int32)
    tblT = jnp.pad(onehot_matrix.T.astype(jnp.float32), ((0, 0), (0, 3)))
    tblT_pad = jnp.pad(tblT.reshape(-1), (0, 24))
    out5 = _encode(seq, tblT_pad)
    return jnp.transpose(out5, (1, 2, 0))
